# baseline (speedup 1.0000x reference)
import functools

import jax
import jax.numpy as jnp
from jax import lax
from jax.experimental import pallas as pl
from jax.experimental.pallas import tpu as pltpu
from jax.experimental.pallas import tpu_sc as plsc

NC = 2
NS = 16
CH = 128
GRP = 10
NBANK = 3


def _sc_mesh():
    return plsc.VectorSubcoreMesh(
        core_axis_name="c", subcore_axis_name="s", num_cores=NC, num_subcores=NS
    )


def _make_deg_kernel(Npad, K):
    rpt = Npad // NS
    Ksc = K // NC

    @functools.partial(
        pl.kernel,
        out_type=jax.ShapeDtypeStruct((NC * Npad, 16), jnp.float32),
        mesh=_sc_mesh(),
        compiler_params=pltpu.CompilerParams(use_tc_tiling_on_sc=False),
        scratch_types=[
            pltpu.VMEM_SHARED((Npad, 16), jnp.float32),
            pltpu.VMEM((K, CH), jnp.int32),
            pltpu.VMEM((CH, 16), jnp.float32),
            pltpu.SemaphoreType.DMA,
        ],
    )
    def deg_kernel(dst_hbm, ones_hbm, zeros_hbm, out_hbm, acc_sh, dst_v, ones_v,
                   ssem):
        c = lax.axis_index("c")
        s = lax.axis_index("s")
        base = s * rpt
        pltpu.sync_copy(zeros_hbm.at[pl.ds(base, rpt)], acc_sh.at[pl.ds(base, rpt)])
        pltpu.sync_copy(dst_hbm.at[s], dst_v)
        pltpu.sync_copy(ones_hbm, ones_v)
        plsc.subcore_barrier()

        @plsc.parallel_loop(0, Ksc, unroll=4)
        def _(j):
            jj = c * Ksc + j
            pltpu.async_copy(ones_v, acc_sh.at[dst_v.at[jj]], ssem, add=True)

        def drain(j, carry):
            pltpu.make_async_copy(ones_v, acc_sh.at[dst_v.at[0]], ssem).wait()
            return carry

        lax.fori_loop(0, Ksc, drain, 0)
        plsc.subcore_barrier()
        pltpu.sync_copy(
            acc_sh.at[pl.ds(base, rpt)],
            out_hbm.at[pl.ds(c * Npad + base, rpt)],
        )

    return deg_kernel


def _make_prop_kernel(B, Npad, K):
    rpt = Npad // NS
    BPC = B // NC
    per_b = [(GRP * i, sz) for i, sz in enumerate(
        [GRP] * (K // GRP) + ([K % GRP] if K % GRP else []))]
    groups = [(b, off, sz) for b in range(BPC) for off, sz in per_b]
    NG = len(groups)

    @functools.partial(
        pl.kernel,
        out_type=jax.ShapeDtypeStruct((B * Npad, 16), jnp.float32),
        mesh=_sc_mesh(),
        compiler_params=pltpu.CompilerParams(use_tc_tiling_on_sc=False),
        scratch_types=[
            pltpu.VMEM_SHARED((BPC, Npad, 16), jnp.float32),
            pltpu.VMEM((BPC, K, CH), jnp.int32),
            pltpu.VMEM((K, CH), jnp.int32),
            pltpu.VMEM((NBANK * GRP, CH, 16), jnp.float32),
            pltpu.SemaphoreType.DMA,
            pltpu.SemaphoreType.DMA,
            pltpu.SemaphoreType.DMA,
        ],
    )
    def prop_kernel(g_hbm, src_hbm, dst_hbm, out_hbm,
                    acc_sh, src_v, dst_v, bufs, gsem, ssem, stsem):
        c = lax.axis_index("c")
        s = lax.axis_index("s")
        base = s * rpt

        for b in range(BPC):
            bb = c * BPC + b
            pltpu.async_copy(
                g_hbm.at[pl.ds(bb * Npad + base, rpt)],
                acc_sh.at[b].at[pl.ds(base, rpt)],
                stsem,
            )
            pltpu.async_copy(src_hbm.at[bb * NS + s], src_v.at[b], stsem)
        pltpu.async_copy(dst_hbm.at[s], dst_v, stsem)
        for b in range(BPC):
            pltpu.make_async_copy(
                g_hbm.at[pl.ds(base, rpt)], acc_sh.at[b].at[pl.ds(base, rpt)],
                stsem).wait()
            pltpu.make_async_copy(src_hbm.at[s], src_v.at[b], stsem).wait()
        pltpu.make_async_copy(dst_hbm.at[s], dst_v, stsem).wait()
        plsc.subcore_barrier()

        def fire_gathers(g, bank):
            b, off, sz = groups[g]

            @plsc.parallel_loop(0, sz, unroll=4)
            def _(i):
                pltpu.async_copy(g_hbm.at[src_v.at[b].at[off + i]],
                                 bufs.at[bank * GRP + i], gsem)

        def drain_gathers(n):
            def drain(i, carry):
                pltpu.make_async_copy(g_hbm.at[src_v.at[0].at[0]], bufs.at[0],
                                      gsem).wait()
                return carry
            lax.fori_loop(0, n, drain, 0)

        def fire_scatters(g, bank):
            b, off, sz = groups[g]

            @plsc.parallel_loop(0, sz, unroll=4)
            def _(i):
                pltpu.async_copy(bufs.at[bank * GRP + i],
                                 acc_sh.at[b].at[dst_v.at[off + i]],
                                 ssem, add=True)

        def drain_scatters(n):
            def drain(i, carry):
                pltpu.make_async_copy(bufs.at[0], acc_sh.at[0].at[dst_v.at[0]],
                                      ssem).wait()
                return carry
            lax.fori_loop(0, n, drain, 0)

        fire_gathers(0, 0)
        for g in range(NG):
            drain_gathers(groups[g][2])
            fire_scatters(g, g % NBANK)
            if g + 1 < NG:
                if g >= NBANK - 1:
                    drain_scatters(groups[g - NBANK + 1][2])
                fire_gathers(g + 1, (g + 1) % NBANK)
        for t in range(max(0, NG - NBANK), NG):
            drain_scatters(groups[t][2])

        plsc.subcore_barrier()

        for b in range(BPC):
            bb = c * BPC + b
            pltpu.async_copy(
                acc_sh.at[b].at[pl.ds(base, rpt)],
                out_hbm.at[pl.ds(bb * Npad + base, rpt)],
                stsem,
            )
        for b in range(BPC):
            pltpu.make_async_copy(
                acc_sh.at[b].at[pl.ds(base, rpt)],
                out_hbm.at[pl.ds(base, rpt)], stsem).wait()

    return prop_kernel


def _dinv_from_deg(deg_ref, N):
    d = deg_ref[...]
    return lax.rsqrt(d[0, :N, :] + d[1, :N, :] + 1.0)


def _tc_first(x, W1, degr, Npad):
    B, N, F = x.shape
    H = W1.shape[1]

    def body(x_ref, w_ref, deg_ref, o_ref):
        dinv = _dinv_from_deg(deg_ref, N)
        h = jnp.dot(x_ref[0], w_ref[...], preferred_element_type=jnp.float32)
        g = dinv * h
        o_ref[0] = jnp.concatenate(
            [g, jnp.zeros((Npad - N, H), jnp.float32)], axis=0)

    return pl.pallas_call(
        body,
        grid=(B,),
        in_specs=[
            pl.BlockSpec((1, N, F), lambda b: (b, 0, 0)),
            pl.BlockSpec((F, H), lambda b: (0, 0)),
            pl.BlockSpec((NC, Npad, H), lambda b: (0, 0, 0)),
        ],
        out_specs=pl.BlockSpec((1, Npad, H), lambda b: (b, 0, 0)),
        out_shape=jax.ShapeDtypeStruct((B, Npad, H), jnp.float32),
    )(x, W1, degr)


def _tc_mid(s1, degr, W2, b1r, N):
    B, Npad, H = s1.shape

    def body(s_ref, deg_ref, w_ref, b_ref, o_ref):
        dinv = _dinv_from_deg(deg_ref, N)
        x1 = jnp.maximum(dinv * s_ref[0, :N, :] + b_ref[...], 0.0)
        h2 = jnp.dot(x1, w_ref[...], preferred_element_type=jnp.float32)
        g = dinv * h2
        o_ref[0] = jnp.concatenate(
            [g, jnp.zeros((Npad - N, H), jnp.float32)], axis=0)

    return pl.pallas_call(
        body,
        grid=(B,),
        in_specs=[
            pl.BlockSpec((1, Npad, H), lambda b: (b, 0, 0)),
            pl.BlockSpec((NC, Npad, H), lambda b: (0, 0, 0)),
            pl.BlockSpec((H, H), lambda b: (0, 0)),
            pl.BlockSpec((1, H), lambda b: (0, 0)),
        ],
        out_specs=pl.BlockSpec((1, Npad, H), lambda b: (b, 0, 0)),
        out_shape=jax.ShapeDtypeStruct((B, Npad, H), jnp.float32),
    )(s1, degr, W2, b1r)


def _tc_final(s2, degr, b2r, Wfc, bfcr, cft, wc1t, bc1t, wc2r, bc2r, N):
    B, Npad, H = s2.shape
    CF, CN = cft.shape

    def body(s_ref, deg_ref, b2_ref, wfc_ref, bfc_ref,
             cft_ref, wc1t_ref, bc1t_ref, wc2r_ref, bc2_ref, o_ref):
        dinv = _dinv_from_deg(deg_ref, N)
        x2 = jnp.maximum(dinv * s_ref[0, :N, :] + b2_ref[...], 0.0)
        nl = jnp.dot(x2, wfc_ref[...],
                     preferred_element_type=jnp.float32)
        hct = jnp.maximum(
            jnp.dot(wc1t_ref[...], cft_ref[...],
                    preferred_element_type=jnp.float32) + bc1t_ref[...],
            0.0,
        )
        clt = jnp.dot(wc2r_ref[...], hct,
                      preferred_element_type=jnp.float32)
        o_ref[0] = (nl + bfc_ref[0, 0]) + (clt + bc2_ref[0, 0])

    return pl.pallas_call(
        body,
        grid=(B,),
        in_specs=[
            pl.BlockSpec((1, Npad, H), lambda b: (b, 0, 0)),
            pl.BlockSpec((NC, Npad, H), lambda b: (0, 0, 0)),
            pl.BlockSpec((1, H), lambda b: (0, 0)),
            pl.BlockSpec((H, 1), lambda b: (0, 0)),
            pl.BlockSpec((1, 1), lambda b: (0, 0)),
            pl.BlockSpec((CF, CN), lambda b: (0, 0)),
            pl.BlockSpec((16, CF), lambda b: (0, 0)),
            pl.BlockSpec((16, 1), lambda b: (0, 0)),
            pl.BlockSpec((1, 16), lambda b: (0, 0)),
            pl.BlockSpec((1, 1), lambda b: (0, 0)),
        ],
        out_specs=pl.BlockSpec((1, N, CN), lambda b: (b, 0, 0)),
        out_shape=jax.ShapeDtypeStruct((B, N, CN), jnp.float32),
    )(s2, degr, b2r, Wfc, bfcr, cft, wc1t, bc1t, wc2r, bc2r)


def kernel(node_features, col_features, edge_index, W1, b1, W2, b2,
           Wfc, bfc, Wc1, bc1, Wc2, bc2):
    B, N, F = node_features.shape
    E = edge_index.shape[1]
    H = W1.shape[1]

    K = -(-E // (NS * CH))
    if K % NC:
        K += 1
    Ep = NS * K * CH
    Npad = (NS * 8) * (-(-(N + 1) // (NS * 8)))

    src = edge_index[0]
    dst = edge_index[1]
    srcp = jnp.concatenate([src, jnp.zeros((Ep - E,), jnp.int32)])
    dstp = jnp.concatenate([dst, jnp.full((Ep - E,), N, jnp.int32)])
    dstp = dstp.reshape(NS, K, CH)
    offs = (jnp.arange(B, dtype=jnp.int32) * Npad)[:, None]
    src_all = (srcp[None, :] + offs).reshape(B * NS, K, CH)

    ones_in = jnp.ones((CH, 16), jnp.float32)
    zeros_in = jnp.zeros((Npad, 16), jnp.float32)

    deg_k = _make_deg_kernel(Npad, K)
    prop_k = _make_prop_kernel(B, Npad, K)

    degp = deg_k(dstp, ones_in, zeros_in)
    degr = degp.reshape(NC, Npad, 16)

    b1r = b1.reshape(1, H)
    b2r = b2.reshape(1, H)
    bfcr = bfc.reshape(1, 1)
    cft = col_features.T
    wc1t = Wc1.T
    bc1t = bc1.reshape(16, 1)
    wc2r = Wc2.reshape(1, 16)
    bc2r = bc2.reshape(1, 1)

    g1 = _tc_first(node_features, W1, degr, Npad)
    s1 = prop_k(g1.reshape(B * Npad, H), src_all, dstp)
    g2 = _tc_mid(s1.reshape(B, Npad, H), degr, W2, b1r, N)
    s2 = prop_k(g2.reshape(B * Npad, H), src_all, dstp)
    out = _tc_final(s2.reshape(B, Npad, H), degr, b2r, Wfc, bfcr,
                    cft, wc1t, bc1t, wc2r, bc2r, N)
    return out.reshape(B, -1)


# speedup vs baseline: 1.0151x; 1.0151x over previous
import functools

import jax
import jax.numpy as jnp
from jax import lax
from jax.experimental import pallas as pl
from jax.experimental.pallas import tpu as pltpu
from jax.experimental.pallas import tpu_sc as plsc

NC = 2
NS = 16
CH = 128
GRP = 16
NBANK = 2


def _sc_mesh():
    return plsc.VectorSubcoreMesh(
        core_axis_name="c", subcore_axis_name="s", num_cores=NC, num_subcores=NS
    )


def _make_deg_kernel(Npad, K):
    rpt = Npad // NS
    Ksc = K // NC

    @functools.partial(
        pl.kernel,
        out_type=jax.ShapeDtypeStruct((NC * Npad, 16), jnp.float32),
        mesh=_sc_mesh(),
        compiler_params=pltpu.CompilerParams(use_tc_tiling_on_sc=False),
        scratch_types=[
            pltpu.VMEM_SHARED((Npad, 16), jnp.float32),
            pltpu.VMEM((K, CH), jnp.int32),
            pltpu.VMEM((CH, 16), jnp.float32),
            pltpu.SemaphoreType.DMA,
        ],
    )
    def deg_kernel(dst_hbm, ones_hbm, zeros_hbm, out_hbm, acc_sh, dst_v, ones_v,
                   ssem):
        c = lax.axis_index("c")
        s = lax.axis_index("s")
        base = s * rpt
        pltpu.sync_copy(zeros_hbm.at[pl.ds(base, rpt)], acc_sh.at[pl.ds(base, rpt)])
        pltpu.sync_copy(dst_hbm.at[s], dst_v)
        pltpu.sync_copy(ones_hbm, ones_v)
        plsc.subcore_barrier()

        @plsc.parallel_loop(0, Ksc, unroll=4)
        def _(j):
            jj = c * Ksc + j
            pltpu.async_copy(ones_v, acc_sh.at[dst_v.at[jj]], ssem, add=True)

        def drain(j, carry):
            pltpu.make_async_copy(ones_v, acc_sh.at[dst_v.at[0]], ssem).wait()
            return carry

        lax.fori_loop(0, Ksc, drain, 0)
        plsc.subcore_barrier()
        pltpu.sync_copy(
            acc_sh.at[pl.ds(base, rpt)],
            out_hbm.at[pl.ds(c * Npad + base, rpt)],
        )

    return deg_kernel


def _make_prop_kernel(B, Npad, K):
    rpt = Npad // NS
    BPC = B // NC
    per_b = [(GRP * i, sz) for i, sz in enumerate(
        [GRP] * (K // GRP) + ([K % GRP] if K % GRP else []))]
    groups = [(b, off, sz) for b in range(BPC) for off, sz in per_b]
    NG = len(groups)

    @functools.partial(
        pl.kernel,
        out_type=jax.ShapeDtypeStruct((B * Npad, 16), jnp.float32),
        mesh=_sc_mesh(),
        compiler_params=pltpu.CompilerParams(use_tc_tiling_on_sc=False),
        scratch_types=[
            pltpu.VMEM_SHARED((BPC, Npad, 16), jnp.float32),
            pltpu.VMEM((BPC, K, CH), jnp.int32),
            pltpu.VMEM((K, CH), jnp.int32),
            pltpu.VMEM((NBANK * GRP, CH, 16), jnp.float32),
            pltpu.SemaphoreType.DMA,
            pltpu.SemaphoreType.DMA,
            pltpu.SemaphoreType.DMA,
        ],
    )
    def prop_kernel(g_hbm, src_hbm, dst_hbm, out_hbm,
                    acc_sh, src_v, dst_v, bufs, gsem, ssem, stsem):
        c = lax.axis_index("c")
        s = lax.axis_index("s")
        base = s * rpt

        for b in range(BPC):
            bb = c * BPC + b
            pltpu.async_copy(
                g_hbm.at[pl.ds(bb * Npad + base, rpt)],
                acc_sh.at[b].at[pl.ds(base, rpt)],
                stsem,
            )
            pltpu.async_copy(src_hbm.at[bb * NS + s], src_v.at[b], stsem)
        pltpu.async_copy(dst_hbm.at[s], dst_v, stsem)
        for b in range(BPC):
            pltpu.make_async_copy(
                g_hbm.at[pl.ds(base, rpt)], acc_sh.at[b].at[pl.ds(base, rpt)],
                stsem).wait()
            pltpu.make_async_copy(src_hbm.at[s], src_v.at[b], stsem).wait()
        pltpu.make_async_copy(dst_hbm.at[s], dst_v, stsem).wait()
        plsc.subcore_barrier()

        def fire_gathers(g, bank):
            b, off, sz = groups[g]

            @plsc.parallel_loop(0, sz, unroll=4)
            def _(i):
                pltpu.async_copy(g_hbm.at[src_v.at[b].at[off + i]],
                                 bufs.at[bank * GRP + i], gsem)

        def drain_gathers(n):
            def drain(i, carry):
                pltpu.make_async_copy(g_hbm.at[src_v.at[0].at[0]], bufs.at[0],
                                      gsem).wait()
                return carry
            lax.fori_loop(0, n, drain, 0)

        def fire_scatters(g, bank):
            b, off, sz = groups[g]

            @plsc.parallel_loop(0, sz, unroll=4)
            def _(i):
                pltpu.async_copy(bufs.at[bank * GRP + i],
                                 acc_sh.at[b].at[dst_v.at[off + i]],
                                 ssem, add=True)

        def drain_scatters(n):
            def drain(i, carry):
                pltpu.make_async_copy(bufs.at[0], acc_sh.at[0].at[dst_v.at[0]],
                                      ssem).wait()
                return carry
            lax.fori_loop(0, n, drain, 0)

        fire_gathers(0, 0)
        for g in range(NG):
            drain_gathers(groups[g][2])
            fire_scatters(g, g % NBANK)
            if g + 1 < NG:
                if g >= NBANK - 1:
                    drain_scatters(groups[g - NBANK + 1][2])
                fire_gathers(g + 1, (g + 1) % NBANK)
        for t in range(max(0, NG - NBANK), NG):
            drain_scatters(groups[t][2])

        plsc.subcore_barrier()

        for b in range(BPC):
            bb = c * BPC + b
            pltpu.async_copy(
                acc_sh.at[b].at[pl.ds(base, rpt)],
                out_hbm.at[pl.ds(bb * Npad + base, rpt)],
                stsem,
            )
        for b in range(BPC):
            pltpu.make_async_copy(
                acc_sh.at[b].at[pl.ds(base, rpt)],
                out_hbm.at[pl.ds(base, rpt)], stsem).wait()

    return prop_kernel


def _dinv_from_deg(deg_ref, N):
    d = deg_ref[...]
    return lax.rsqrt(d[0, :N, :] + d[1, :N, :] + 1.0)


def _tc_first(x, W1, degr, Npad):
    B, N, F = x.shape
    H = W1.shape[1]

    def body(x_ref, w_ref, deg_ref, o_ref):
        dinv = _dinv_from_deg(deg_ref, N)
        h = jnp.dot(x_ref[0], w_ref[...], preferred_element_type=jnp.float32)
        g = dinv * h
        o_ref[0] = jnp.concatenate(
            [g, jnp.zeros((Npad - N, H), jnp.float32)], axis=0)

    return pl.pallas_call(
        body,
        grid=(B,),
        in_specs=[
            pl.BlockSpec((1, N, F), lambda b: (b, 0, 0)),
            pl.BlockSpec((F, H), lambda b: (0, 0)),
            pl.BlockSpec((NC, Npad, H), lambda b: (0, 0, 0)),
        ],
        out_specs=pl.BlockSpec((1, Npad, H), lambda b: (b, 0, 0)),
        out_shape=jax.ShapeDtypeStruct((B, Npad, H), jnp.float32),
    )(x, W1, degr)


def _tc_mid(s1, degr, W2, b1r, N):
    B, Npad, H = s1.shape

    def body(s_ref, deg_ref, w_ref, b_ref, o_ref):
        dinv = _dinv_from_deg(deg_ref, N)
        x1 = jnp.maximum(dinv * s_ref[0, :N, :] + b_ref[...], 0.0)
        h2 = jnp.dot(x1, w_ref[...], preferred_element_type=jnp.float32)
        g = dinv * h2
        o_ref[0] = jnp.concatenate(
            [g, jnp.zeros((Npad - N, H), jnp.float32)], axis=0)

    return pl.pallas_call(
        body,
        grid=(B,),
        in_specs=[
            pl.BlockSpec((1, Npad, H), lambda b: (b, 0, 0)),
            pl.BlockSpec((NC, Npad, H), lambda b: (0, 0, 0)),
            pl.BlockSpec((H, H), lambda b: (0, 0)),
            pl.BlockSpec((1, H), lambda b: (0, 0)),
        ],
        out_specs=pl.BlockSpec((1, Npad, H), lambda b: (b, 0, 0)),
        out_shape=jax.ShapeDtypeStruct((B, Npad, H), jnp.float32),
    )(s1, degr, W2, b1r)


def _tc_final(s2, degr, b2r, Wfc, bfcr, cft, wc1t, bc1t, wc2r, bc2r, N):
    B, Npad, H = s2.shape
    CF, CN = cft.shape

    def body(s_ref, deg_ref, b2_ref, wfc_ref, bfc_ref,
             cft_ref, wc1t_ref, bc1t_ref, wc2r_ref, bc2_ref, o_ref):
        dinv = _dinv_from_deg(deg_ref, N)
        x2 = jnp.maximum(dinv * s_ref[0, :N, :] + b2_ref[...], 0.0)
        nl = jnp.dot(x2, wfc_ref[...],
                     preferred_element_type=jnp.float32)
        hct = jnp.maximum(
            jnp.dot(wc1t_ref[...], cft_ref[...],
                    preferred_element_type=jnp.float32) + bc1t_ref[...],
            0.0,
        )
        clt = jnp.dot(wc2r_ref[...], hct,
                      preferred_element_type=jnp.float32)
        o_ref[0] = (nl + bfc_ref[0, 0]) + (clt + bc2_ref[0, 0])

    return pl.pallas_call(
        body,
        grid=(B,),
        in_specs=[
            pl.BlockSpec((1, Npad, H), lambda b: (b, 0, 0)),
            pl.BlockSpec((NC, Npad, H), lambda b: (0, 0, 0)),
            pl.BlockSpec((1, H), lambda b: (0, 0)),
            pl.BlockSpec((H, 1), lambda b: (0, 0)),
            pl.BlockSpec((1, 1), lambda b: (0, 0)),
            pl.BlockSpec((CF, CN), lambda b: (0, 0)),
            pl.BlockSpec((16, CF), lambda b: (0, 0)),
            pl.BlockSpec((16, 1), lambda b: (0, 0)),
            pl.BlockSpec((1, 16), lambda b: (0, 0)),
            pl.BlockSpec((1, 1), lambda b: (0, 0)),
        ],
        out_specs=pl.BlockSpec((1, N, CN), lambda b: (b, 0, 0)),
        out_shape=jax.ShapeDtypeStruct((B, N, CN), jnp.float32),
    )(s2, degr, b2r, Wfc, bfcr, cft, wc1t, bc1t, wc2r, bc2r)


def kernel(node_features, col_features, edge_index, W1, b1, W2, b2,
           Wfc, bfc, Wc1, bc1, Wc2, bc2):
    B, N, F = node_features.shape
    E = edge_index.shape[1]
    H = W1.shape[1]

    K = -(-E // (NS * CH))
    if K % NC:
        K += 1
    Ep = NS * K * CH
    Npad = (NS * 8) * (-(-(N + 1) // (NS * 8)))

    src = edge_index[0]
    dst = edge_index[1]
    srcp = jnp.concatenate([src, jnp.zeros((Ep - E,), jnp.int32)])
    dstp = jnp.concatenate([dst, jnp.full((Ep - E,), N, jnp.int32)])
    dstp = dstp.reshape(NS, K, CH)
    offs = (jnp.arange(B, dtype=jnp.int32) * Npad)[:, None]
    src_all = (srcp[None, :] + offs).reshape(B * NS, K, CH)

    ones_in = jnp.ones((CH, 16), jnp.float32)
    zeros_in = jnp.zeros((Npad, 16), jnp.float32)

    deg_k = _make_deg_kernel(Npad, K)
    prop_k = _make_prop_kernel(B, Npad, K)

    degp = deg_k(dstp, ones_in, zeros_in)
    degr = degp.reshape(NC, Npad, 16)

    b1r = b1.reshape(1, H)
    b2r = b2.reshape(1, H)
    bfcr = bfc.reshape(1, 1)
    cft = col_features.T
    wc1t = Wc1.T
    bc1t = bc1.reshape(16, 1)
    wc2r = Wc2.reshape(1, 16)
    bc2r = bc2.reshape(1, 1)

    g1 = _tc_first(node_features, W1, degr, Npad)
    s1 = prop_k(g1.reshape(B * Npad, H), src_all, dstp)
    g2 = _tc_mid(s1.reshape(B, Npad, H), degr, W2, b1r, N)
    s2 = prop_k(g2.reshape(B * Npad, H), src_all, dstp)
    out = _tc_final(s2.reshape(B, Npad, H), degr, b2r, Wfc, bfcr,
                    cft, wc1t, bc1t, wc2r, bc2r, N)
    return out.reshape(B, -1)
